# baseline (speedup 1.0000x reference)
import functools

import jax
import jax.numpy as jnp
from jax import lax
from jax.experimental import pallas as pl
from jax.experimental.pallas import tpu as pltpu
from jax.experimental.pallas import tpu_sc as plsc

D = 768
ACCW = 896
NC = 2
NS = 16
NW = NC * NS
GCH = 16
SCH = 32


def _mlp_body(x_ref, w1_ref, b1_ref, w2_ref, b2_ref, o_ref, *, nblk):
    i = pl.program_id(0)

    @pl.when(i < nblk)
    def _():
        bf16 = jnp.bfloat16
        x = x_ref[...].astype(bf16)
        h = lax.dot_general(x, w1_ref[...].astype(bf16),
                            (((1,), (1,)), ((), ())),
                            preferred_element_type=jnp.float32)
        h = h + b1_ref[...]
        h = 0.5 * h * (1.0 + lax.erf(h * 0.7071067811865476))
        o = lax.dot_general(h.astype(bf16), w2_ref[...].astype(bf16),
                            (((1,), (1,)), ((), ())),
                            preferred_element_type=jnp.float32)
        o_ref[...] = o + b2_ref[...]

    @pl.when(i >= nblk)
    def _():
        o_ref[...] = jnp.zeros_like(o_ref)


def _mlp(x, w1, b1, w2, b2, extra_zero_blocks=0, blk=256):
    n = x.shape[0]
    nblk = n // blk
    grid = (nblk + extra_zero_blocks,)
    out = pl.pallas_call(
        functools.partial(_mlp_body, nblk=nblk),
        grid=grid,
        in_specs=[
            pl.BlockSpec((blk, D), lambda i: (jnp.minimum(i, nblk - 1), 0)),
            pl.BlockSpec((D, D), lambda i: (0, 0)),
            pl.BlockSpec((1, D), lambda i: (0, 0)),
            pl.BlockSpec((D, D), lambda i: (0, 0)),
            pl.BlockSpec((1, D), lambda i: (0, 0)),
        ],
        out_specs=pl.BlockSpec((blk, D), lambda i: (i, 0)),
        out_shape=jax.ShapeDtypeStruct(((nblk + extra_zero_blocks) * blk, D),
                                       jnp.float32),
    )(x, w1, b1.reshape(1, D), w2, b2.reshape(1, D))
    return out


def _ln(x, g, b):
    mu = jnp.mean(x, axis=-1, keepdims=True)
    var = jnp.mean((x - mu) ** 2, axis=-1, keepdims=True)
    return (x - mu) * lax.rsqrt(var + 1e-5) * g + b


def _fin_jamo_body(jamo_ref, gath_ref, g_ref, b_ref, o_ref):
    x = jamo_ref[...] + gath_ref[...]
    o_ref[...] = _ln(x, g_ref[...], b_ref[...])


def _fin_jamo(jamo, gath, g, b, blk=256):
    n = jamo.shape[0]
    return pl.pallas_call(
        _fin_jamo_body,
        grid=(n // blk,),
        in_specs=[
            pl.BlockSpec((blk, D), lambda i: (i, 0)),
            pl.BlockSpec((blk, D), lambda i: (i, 0)),
            pl.BlockSpec((1, D), lambda i: (0, 0)),
            pl.BlockSpec((1, D), lambda i: (0, 0)),
        ],
        out_specs=pl.BlockSpec((blk, D), lambda i: (i, 0)),
        out_shape=jax.ShapeDtypeStruct((n, D), jnp.float32),
    )(jamo, gath, g.reshape(1, D), b.reshape(1, D))


def _fin_syll_body(acc_ref, syll_ref, g_ref, b_ref, o_ref):
    a = acc_ref[...]
    sums = a[:, :D]
    cnt = jnp.sum(a[:, D:], axis=1, keepdims=True)
    mean = jnp.where(cnt > 0, sums / jnp.maximum(cnt, 1.0), 0.0)
    x = syll_ref[...] + mean
    o_ref[...] = _ln(x, g_ref[...], b_ref[...])


def _fin_syll(acc, syll, g, b, blk=256):
    n = syll.shape[0]
    return pl.pallas_call(
        _fin_syll_body,
        grid=(n // blk,),
        in_specs=[
            pl.BlockSpec((blk, ACCW), lambda i: (i, 0)),
            pl.BlockSpec((blk, D), lambda i: (i, 0)),
            pl.BlockSpec((1, D), lambda i: (0, 0)),
            pl.BlockSpec((1, D), lambda i: (0, 0)),
        ],
        out_specs=pl.BlockSpec((blk, D), lambda i: (i, 0)),
        out_shape=jax.ShapeDtypeStruct((n, D), jnp.float32),
    )(acc, syll, g.reshape(1, D), b.reshape(1, D))


def _sc_body(table, jc, gidx, sidx, gath_out, acc_out,
             idx256, rowsg, sidxv, rows2, acc,
             cnt_sm, semg0, semg1, semw0, semw1, *,
             per_w, sj, nbatch_per_core, segs_per_tile):
    c = lax.axis_index("c")
    s = lax.axis_index("s")
    wid = c * NS + s
    base = pl.multiple_of(wid * per_w, per_w)
    i32 = jnp.int32
    semg = [semg0, semg1]
    semw = [semw0, semw1]

    ngc = per_w // GCH
    pltpu.sync_copy(gidx.at[pl.ds(base, per_w)], idx256)

    def _gstart(ch, par):
        return pltpu.async_copy(table.at[idx256.at[pl.ds(ch * GCH, GCH)]],
                                rowsg.at[par], semg[par])

    wbs = [None, None]
    cur = _gstart(0, 0)
    for ch in range(ngc):
        par = ch % 2
        cur.wait()
        if ch + 1 < ngc:
            if wbs[1 - par] is not None:
                wbs[1 - par].wait()
            cur = _gstart(ch + 1, 1 - par)
        off = pl.multiple_of(base + ch * GCH, GCH)
        wbs[par] = pltpu.async_copy(rowsg.at[par],
                                    gath_out.at[pl.ds(off, GCH)], semw[par])
    for wb in wbs:
        if wb is not None:
            wb.wait()

    tiles_per_batch = NS // nbatch_per_core
    b = nbatch_per_core * c + s // tiles_per_batch
    band = (s % tiles_per_batch) * segs_per_tile
    lo = band
    hi = band + segs_per_tile
    bbase = pl.multiple_of(b * sj, sj)

    zero = jnp.zeros((16,), jnp.float32)

    def _zrow(i, carry):
        for cb in range(ACCW // 16):
            acc[i, pl.ds(cb * 16, 16)] = zero
        return carry

    lax.fori_loop(0, segs_per_tile, _zrow, 0)

    pltpu.sync_copy(sidx.at[pl.ds(bbase, sj)], sidxv.at[pl.ds(0, sj)])

    lane_iota = lax.iota(i32, 16)

    cnt_sm[0] = 0
    cnt_sm[1] = 0

    def _count(i, carry):
        v = sidxv[pl.ds(i * 16, 16)]
        for j, bound in ((0, lo), (1, hi)):
            blt = v < bound
            all_lt = jnp.all(blt)
            any_lt = jnp.any(blt)

            @pl.when(all_lt)
            def _(j=j):
                cnt_sm[j] = cnt_sm[j] + 16

            @pl.when(any_lt & jnp.logical_not(all_lt))
            def _(j=j, blt=blt):
                t = cnt_sm[j]
                for lane in range(16):
                    t = t + jnp.where(
                        jnp.any(blt & (lane_iota == lane)), 1, 0).astype(i32)
                cnt_sm[j] = t
        return carry

    lax.fori_loop(0, sj // 16, _count, 0)
    start = cnt_sm[0]
    end = cnt_sm[1]
    start16 = (start // 16) * 16

    def _lane_splat(v, lane):
        idx = jnp.broadcast_to(lane.astype(i32), (16,))
        dnums = lax.GatherDimensionNumbers(
            offset_dims=(), collapsed_slice_dims=(0,), start_index_map=(0,))
        return lax.gather(v, idx[:, None], dnums, (1,),
                          mode=lax.GatherScatterMode.PROMISE_IN_BOUNDS)

    one0 = jnp.where(lane_iota == 0, 1.0, 0.0)

    def _chunk(k, carry):
        p0 = pl.multiple_of(start16 + k * SCH, 16)
        pltpu.sync_copy(jc.at[pl.ds(bbase + p0, SCH)], rows2)
        for r in range(SCH):
            p = p0 + r

            @pl.when((p >= start) & (p < end))
            def _(p=p, r=r):
                v = sidxv[pl.ds((p // 16) * 16, 16)]
                row = _lane_splat(v, p % 16) - lo
                for cb in range(D // 16):
                    plsc.addupdate_scatter(
                        acc, [row, cb * 16 + lane_iota],
                        rows2[r, pl.ds(cb * 16, 16)])
                plsc.addupdate_scatter(acc, [row, D + lane_iota], one0)
        return carry

    nch = (end - start16 + SCH - 1) // SCH
    lax.fori_loop(0, nch, _chunk, 0)

    out_off = pl.multiple_of(
        b * (segs_per_tile * tiles_per_batch) + band, segs_per_tile)
    pltpu.sync_copy(acc, acc_out.at[pl.ds(out_off, segs_per_tile)])


def _sc_gather_segsum(table, jc, gidx, sidx, nseg, sj):
    nj = gidx.shape[0]
    per_w = nj // NW
    nbatch_per_core = (nj // sj) // NC
    segs_per_tile = nseg // (nj // sj) // (NS // nbatch_per_core)
    mesh = plsc.VectorSubcoreMesh(core_axis_name="c", subcore_axis_name="s",
                                  num_cores=NC, num_subcores=NS)
    kern = pl.kernel(
        functools.partial(_sc_body, per_w=per_w, sj=sj,
                          nbatch_per_core=nbatch_per_core,
                          segs_per_tile=segs_per_tile),
        out_type=(
            jax.ShapeDtypeStruct((nj, D), jnp.float32),
            jax.ShapeDtypeStruct((nseg, ACCW), jnp.float32),
        ),
        mesh=mesh,
        compiler_params=pltpu.CompilerParams(use_tc_tiling_on_sc=False,
                                             needs_layout_passes=False),
        scratch_types=[
            pltpu.VMEM((per_w,), jnp.int32),
            pltpu.VMEM((2, GCH, D), jnp.float32),
            pltpu.VMEM((sj + 32,), jnp.int32),
            pltpu.VMEM((SCH, D), jnp.float32),
            pltpu.VMEM((segs_per_tile, ACCW), jnp.float32),
            pltpu.SMEM((2,), jnp.int32),
            pltpu.SemaphoreType.DMA,
            pltpu.SemaphoreType.DMA,
            pltpu.SemaphoreType.DMA,
            pltpu.SemaphoreType.DMA,
        ],
    )
    return kern(table, jc, gidx, sidx)


def kernel(jamo_features, syllable_features, syllable_indices,
           W1a, b1a, W2a, b2a, W1b, b1b, W2b, b2b, g1, beta1, g2, beta2):
    B, SJ, _ = jamo_features.shape
    _, SS, _ = syllable_features.shape
    nj = B * SJ
    nseg = B * SS

    jamo2 = jamo_features.reshape(nj, D)
    syll2 = syllable_features.reshape(nseg, D)

    table = _mlp(syll2, W1a, b1a, W2a, b2a, extra_zero_blocks=1)
    jc = _mlp(jamo2, W1b, b1b, W2b, b2b, extra_zero_blocks=1)

    si = syllable_indices.astype(jnp.int32)
    valid = (si >= 1) & (si <= SS)
    brow = (jnp.arange(B, dtype=jnp.int32) * SS)[:, None]
    gidx = jnp.where(valid, brow + si - 1, nseg).reshape(nj)
    sidx = jnp.where(valid, si - 1, -1).reshape(nj)

    gath, acc = _sc_gather_segsum(table, jc, gidx, sidx, nseg, SJ)

    out1 = _fin_jamo(jamo2, gath, g1, beta1)
    out2 = _fin_syll(acc, syll2, g2, beta2)
    return (out1.reshape(B, SJ, D), out2.reshape(B, SS, D))


# speedup vs baseline: 1.2374x; 1.2374x over previous
import functools

import jax
import jax.numpy as jnp
from jax import lax
from jax.experimental import pallas as pl
from jax.experimental.pallas import tpu as pltpu
from jax.experimental.pallas import tpu_sc as plsc

D = 768
ACCW = 896
NC = 2
NS = 16
NW = NC * NS
GCH = 16
SCH = 32


def _mlp_body(x_ref, w1_ref, b1_ref, w2_ref, b2_ref, o_ref, *, nblk):
    i = pl.program_id(0)

    @pl.when(i < nblk)
    def _():
        bf16 = jnp.bfloat16
        x = x_ref[...].astype(bf16)
        h = lax.dot_general(x, w1_ref[...].astype(bf16),
                            (((1,), (1,)), ((), ())),
                            preferred_element_type=jnp.float32)
        h = h + b1_ref[...]
        h = 0.5 * h * (1.0 + lax.erf(h * 0.7071067811865476))
        o = lax.dot_general(h.astype(bf16), w2_ref[...].astype(bf16),
                            (((1,), (1,)), ((), ())),
                            preferred_element_type=jnp.float32)
        o_ref[...] = o + b2_ref[...]

    @pl.when(i >= nblk)
    def _():
        o_ref[...] = jnp.zeros_like(o_ref)


def _mlp(x, w1, b1, w2, b2, extra_zero_blocks=0, blk=256):
    n = x.shape[0]
    nblk = n // blk
    grid = (nblk + extra_zero_blocks,)
    out = pl.pallas_call(
        functools.partial(_mlp_body, nblk=nblk),
        grid=grid,
        in_specs=[
            pl.BlockSpec((blk, D), lambda i: (jnp.minimum(i, nblk - 1), 0)),
            pl.BlockSpec((D, D), lambda i: (0, 0)),
            pl.BlockSpec((1, D), lambda i: (0, 0)),
            pl.BlockSpec((D, D), lambda i: (0, 0)),
            pl.BlockSpec((1, D), lambda i: (0, 0)),
        ],
        out_specs=pl.BlockSpec((blk, D), lambda i: (i, 0)),
        out_shape=jax.ShapeDtypeStruct(((nblk + extra_zero_blocks) * blk, D),
                                       jnp.float32),
    )(x, w1, b1.reshape(1, D), w2, b2.reshape(1, D))
    return out


def _ln(x, g, b):
    mu = jnp.mean(x, axis=-1, keepdims=True)
    var = jnp.mean((x - mu) ** 2, axis=-1, keepdims=True)
    return (x - mu) * lax.rsqrt(var + 1e-5) * g + b


def _fin_jamo_body(jamo_ref, gath_ref, g_ref, b_ref, o_ref):
    x = jamo_ref[...] + gath_ref[...]
    o_ref[...] = _ln(x, g_ref[...], b_ref[...])


def _fin_jamo(jamo, gath, g, b, blk=256):
    n = jamo.shape[0]
    return pl.pallas_call(
        _fin_jamo_body,
        grid=(n // blk,),
        in_specs=[
            pl.BlockSpec((blk, D), lambda i: (i, 0)),
            pl.BlockSpec((blk, D), lambda i: (i, 0)),
            pl.BlockSpec((1, D), lambda i: (0, 0)),
            pl.BlockSpec((1, D), lambda i: (0, 0)),
        ],
        out_specs=pl.BlockSpec((blk, D), lambda i: (i, 0)),
        out_shape=jax.ShapeDtypeStruct((n, D), jnp.float32),
    )(jamo, gath, g.reshape(1, D), b.reshape(1, D))


def _fin_syll_body(acc_ref, syll_ref, g_ref, b_ref, o_ref):
    a = acc_ref[...]
    sums = a[:, :D]
    cnt = jnp.sum(a[:, D:], axis=1, keepdims=True)
    mean = jnp.where(cnt > 0, sums / jnp.maximum(cnt, 1.0), 0.0)
    x = syll_ref[...] + mean
    o_ref[...] = _ln(x, g_ref[...], b_ref[...])


def _fin_syll(acc, syll, g, b, blk=256):
    n = syll.shape[0]
    return pl.pallas_call(
        _fin_syll_body,
        grid=(n // blk,),
        in_specs=[
            pl.BlockSpec((blk, ACCW), lambda i: (i, 0)),
            pl.BlockSpec((blk, D), lambda i: (i, 0)),
            pl.BlockSpec((1, D), lambda i: (0, 0)),
            pl.BlockSpec((1, D), lambda i: (0, 0)),
        ],
        out_specs=pl.BlockSpec((blk, D), lambda i: (i, 0)),
        out_shape=jax.ShapeDtypeStruct((n, D), jnp.float32),
    )(acc, syll, g.reshape(1, D), b.reshape(1, D))


def _sc_body(table, jc, gidx, sidx, gath_out, acc_out,
             idx256, rowsg, sidxv, rows2, acc,
             cnt_sm, semg0, semg1, semw0, semw1, *,
             per_w, sj, nbatch_per_core, segs_per_tile):
    c = lax.axis_index("c")
    s = lax.axis_index("s")
    wid = c * NS + s
    base = pl.multiple_of(wid * per_w, per_w)
    i32 = jnp.int32
    semg = [semg0, semg1]
    semw = [semw0, semw1]

    ngc = per_w // GCH
    pltpu.sync_copy(gidx.at[pl.ds(base, per_w)], idx256)

    def _gstart(ch, par):
        return pltpu.async_copy(table.at[idx256.at[pl.ds(ch * GCH, GCH)]],
                                rowsg.at[par], semg[par])

    wbs = [None, None]
    cur = _gstart(0, 0)
    for ch in range(ngc):
        par = ch % 2
        cur.wait()
        if ch + 1 < ngc:
            if wbs[1 - par] is not None:
                wbs[1 - par].wait()
            cur = _gstart(ch + 1, 1 - par)
        off = pl.multiple_of(base + ch * GCH, GCH)
        wbs[par] = pltpu.async_copy(rowsg.at[par],
                                    gath_out.at[pl.ds(off, GCH)], semw[par])
    for wb in wbs:
        if wb is not None:
            wb.wait()

    tiles_per_batch = NS // nbatch_per_core
    b = nbatch_per_core * c + s // tiles_per_batch
    band = (s % tiles_per_batch) * segs_per_tile
    lo = band
    hi = band + segs_per_tile
    bbase = pl.multiple_of(b * sj, sj)

    zero = jnp.zeros((16,), jnp.float32)

    def _zrow(i, carry):
        for cb in range(ACCW // 16):
            acc[i, pl.ds(cb * 16, 16)] = zero
        return carry

    lax.fori_loop(0, segs_per_tile, _zrow, 0)

    pltpu.sync_copy(sidx.at[pl.ds(bbase, sj)], sidxv.at[pl.ds(0, sj)])

    lane_iota = lax.iota(i32, 16)

    cnt_sm[0] = 0
    cnt_sm[1] = 0

    def _count(i, carry):
        v = sidxv[pl.ds(i * 16, 16)]
        for j, bound in ((0, lo), (1, hi)):
            blt = v < bound
            all_lt = jnp.all(blt)
            any_lt = jnp.any(blt)

            @pl.when(all_lt)
            def _(j=j):
                cnt_sm[j] = cnt_sm[j] + 16

            @pl.when(any_lt & jnp.logical_not(all_lt))
            def _(j=j, blt=blt):
                t = cnt_sm[j]
                for lane in range(16):
                    t = t + jnp.where(
                        jnp.any(blt & (lane_iota == lane)), 1, 0).astype(i32)
                cnt_sm[j] = t
        return carry

    lax.fori_loop(0, sj // 16, _count, 0)
    start = cnt_sm[0]
    end = cnt_sm[1]
    start16 = (start // 16) * 16

    def _lane_splat(v, lane):
        idx = jnp.broadcast_to(lane.astype(i32), (16,))
        dnums = lax.GatherDimensionNumbers(
            offset_dims=(), collapsed_slice_dims=(0,), start_index_map=(0,))
        return lax.gather(v, idx[:, None], dnums, (1,),
                          mode=lax.GatherScatterMode.PROMISE_IN_BOUNDS)

    one0 = jnp.where(lane_iota == 0, 1.0, 0.0)

    def _chunk(k, carry):
        p0 = pl.multiple_of(start16 + k * SCH, 16)
        pltpu.sync_copy(jc.at[pl.ds(bbase + p0, SCH)], rows2)
        for r in range(SCH):
            p = p0 + r

            @pl.when((p >= start) & (p < end))
            def _(p=p, r=r):
                v = sidxv[pl.ds((p // 16) * 16, 16)]
                row = _lane_splat(v, p % 16) - lo
                for cb in range(D // 16):
                    plsc.addupdate_scatter(
                        acc, [row, cb * 16 + lane_iota],
                        rows2[r, pl.ds(cb * 16, 16)])
                plsc.addupdate_scatter(acc, [row, D + lane_iota], one0)
        return carry

    nch = (end - start16 + SCH - 1) // SCH
    lax.fori_loop(0, nch, _chunk, 0)

    out_off = pl.multiple_of(
        b * (segs_per_tile * tiles_per_batch) + band, segs_per_tile)
    pltpu.sync_copy(acc, acc_out.at[pl.ds(out_off, segs_per_tile)])


def _sc_gather_segsum(table, jc, gidx, sidx, nseg, sj):
    nj = gidx.shape[0]
    per_w = nj // NW
    nbatch_per_core = (nj // sj) // NC
    segs_per_tile = nseg // (nj // sj) // (NS // nbatch_per_core)
    mesh = plsc.VectorSubcoreMesh(core_axis_name="c", subcore_axis_name="s",
                                  num_cores=NC, num_subcores=NS)
    kern = pl.kernel(
        functools.partial(_sc_body, per_w=per_w, sj=sj,
                          nbatch_per_core=nbatch_per_core,
                          segs_per_tile=segs_per_tile),
        out_type=(
            jax.ShapeDtypeStruct((nj, D), jnp.float32),
            jax.ShapeDtypeStruct((nseg, ACCW), jnp.float32),
        ),
        mesh=mesh,
        compiler_params=pltpu.CompilerParams(use_tc_tiling_on_sc=True,
                                             needs_layout_passes=False),
        scratch_types=[
            pltpu.VMEM((per_w,), jnp.int32),
            pltpu.VMEM((2, GCH, D), jnp.float32),
            pltpu.VMEM((sj + 32,), jnp.int32),
            pltpu.VMEM((SCH, D), jnp.float32),
            pltpu.VMEM((segs_per_tile, ACCW), jnp.float32),
            pltpu.SMEM((2,), jnp.int32),
            pltpu.SemaphoreType.DMA,
            pltpu.SemaphoreType.DMA,
            pltpu.SemaphoreType.DMA,
            pltpu.SemaphoreType.DMA,
        ],
    )
    return kern(table, jc, gidx, sidx)


def kernel(jamo_features, syllable_features, syllable_indices,
           W1a, b1a, W2a, b2a, W1b, b1b, W2b, b2b, g1, beta1, g2, beta2):
    B, SJ, _ = jamo_features.shape
    _, SS, _ = syllable_features.shape
    nj = B * SJ
    nseg = B * SS

    jamo2 = jamo_features.reshape(nj, D)
    syll2 = syllable_features.reshape(nseg, D)

    table = _mlp(syll2, W1a, b1a, W2a, b2a, extra_zero_blocks=1)
    jc = _mlp(jamo2, W1b, b1b, W2b, b2b, extra_zero_blocks=1)

    si = syllable_indices.astype(jnp.int32)
    valid = (si >= 1) & (si <= SS)
    brow = (jnp.arange(B, dtype=jnp.int32) * SS)[:, None]
    gidx = jnp.where(valid, brow + si - 1, nseg).reshape(nj)
    sidx = jnp.where(valid, si - 1, -1).reshape(nj)

    gath, acc = _sc_gather_segsum(table, jc, gidx, sidx, nseg, SJ)

    out1 = _fin_jamo(jamo2, gath, g1, beta1)
    out2 = _fin_syll(acc, syll2, g2, beta2)
    return (out1.reshape(B, SJ, D), out2.reshape(B, SS, D))


# speedup vs baseline: 1.5512x; 1.2536x over previous
import functools

import jax
import jax.numpy as jnp
from jax import lax
from jax.experimental import pallas as pl
from jax.experimental.pallas import tpu as pltpu
from jax.experimental.pallas import tpu_sc as plsc

D = 768
ACCW = 896
NC = 2
NS = 16
NW = NC * NS
GCH = 16
SCH = 32


def _mlp_body(x_ref, w1_ref, b1_ref, w2_ref, b2_ref, o_ref, *, nblk):
    i = pl.program_id(0)

    @pl.when(i < nblk)
    def _():
        bf16 = jnp.bfloat16
        x = x_ref[...].astype(bf16)
        h = lax.dot_general(x, w1_ref[...].astype(bf16),
                            (((1,), (1,)), ((), ())),
                            preferred_element_type=jnp.float32)
        h = h + b1_ref[...]
        h = 0.5 * h * (1.0 + lax.erf(h * 0.7071067811865476))
        o = lax.dot_general(h.astype(bf16), w2_ref[...].astype(bf16),
                            (((1,), (1,)), ((), ())),
                            preferred_element_type=jnp.float32)
        o_ref[...] = o + b2_ref[...]

    @pl.when(i >= nblk)
    def _():
        o_ref[...] = jnp.zeros_like(o_ref)


def _mlp(x, w1, b1, w2, b2, extra_zero_blocks=0, blk=256):
    n = x.shape[0]
    nblk = n // blk
    grid = (nblk + extra_zero_blocks,)
    out = pl.pallas_call(
        functools.partial(_mlp_body, nblk=nblk),
        grid=grid,
        in_specs=[
            pl.BlockSpec((blk, D), lambda i: (jnp.minimum(i, nblk - 1), 0)),
            pl.BlockSpec((D, D), lambda i: (0, 0)),
            pl.BlockSpec((1, D), lambda i: (0, 0)),
            pl.BlockSpec((D, D), lambda i: (0, 0)),
            pl.BlockSpec((1, D), lambda i: (0, 0)),
        ],
        out_specs=pl.BlockSpec((blk, D), lambda i: (i, 0)),
        out_shape=jax.ShapeDtypeStruct(((nblk + extra_zero_blocks) * blk, D),
                                       jnp.float32),
    )(x, w1, b1.reshape(1, D), w2, b2.reshape(1, D))
    return out


def _ln(x, g, b):
    mu = jnp.mean(x, axis=-1, keepdims=True)
    var = jnp.mean((x - mu) ** 2, axis=-1, keepdims=True)
    return (x - mu) * lax.rsqrt(var + 1e-5) * g + b


def _fin_jamo_body(jamo_ref, gath_ref, g_ref, b_ref, o_ref):
    x = jamo_ref[...] + gath_ref[...]
    o_ref[...] = _ln(x, g_ref[...], b_ref[...])


def _fin_jamo(jamo, gath, g, b, blk=256):
    n = jamo.shape[0]
    return pl.pallas_call(
        _fin_jamo_body,
        grid=(n // blk,),
        in_specs=[
            pl.BlockSpec((blk, D), lambda i: (i, 0)),
            pl.BlockSpec((blk, D), lambda i: (i, 0)),
            pl.BlockSpec((1, D), lambda i: (0, 0)),
            pl.BlockSpec((1, D), lambda i: (0, 0)),
        ],
        out_specs=pl.BlockSpec((blk, D), lambda i: (i, 0)),
        out_shape=jax.ShapeDtypeStruct((n, D), jnp.float32),
    )(jamo, gath, g.reshape(1, D), b.reshape(1, D))


def _fin_syll_body(acc_ref, syll_ref, g_ref, b_ref, o_ref):
    a = acc_ref[...]
    sums = a[:, :D]
    cnt = jnp.sum(a[:, D:], axis=1, keepdims=True)
    mean = jnp.where(cnt > 0, sums / jnp.maximum(cnt, 1.0), 0.0)
    x = syll_ref[...] + mean
    o_ref[...] = _ln(x, g_ref[...], b_ref[...])


def _fin_syll(acc, syll, g, b, blk=256):
    n = syll.shape[0]
    return pl.pallas_call(
        _fin_syll_body,
        grid=(n // blk,),
        in_specs=[
            pl.BlockSpec((blk, ACCW), lambda i: (i, 0)),
            pl.BlockSpec((blk, D), lambda i: (i, 0)),
            pl.BlockSpec((1, D), lambda i: (0, 0)),
            pl.BlockSpec((1, D), lambda i: (0, 0)),
        ],
        out_specs=pl.BlockSpec((blk, D), lambda i: (i, 0)),
        out_shape=jax.ShapeDtypeStruct((n, D), jnp.float32),
    )(acc, syll, g.reshape(1, D), b.reshape(1, D))


def _sc_gather_body(table, gidx, gath_out,
                    idx256, rowsg, semg0, semg1, semw0, semw1, *, per_w):
    c = lax.axis_index("c")
    s = lax.axis_index("s")
    wid = c * NS + s
    base = pl.multiple_of(wid * per_w, per_w)
    semg = [semg0, semg1]
    semw = [semw0, semw1]

    ngc = per_w // GCH
    pltpu.sync_copy(gidx.at[pl.ds(base, per_w)], idx256)

    def _gstart(ch, par):
        return pltpu.async_copy(table.at[idx256.at[pl.ds(ch * GCH, GCH)]],
                                rowsg.at[par], semg[par])

    wbs = [None, None]
    cur = _gstart(0, 0)
    for ch in range(ngc):
        par = ch % 2
        cur.wait()
        if ch + 1 < ngc:
            if wbs[1 - par] is not None:
                wbs[1 - par].wait()
            cur = _gstart(ch + 1, 1 - par)
        off = pl.multiple_of(base + ch * GCH, GCH)
        wbs[par] = pltpu.async_copy(rowsg.at[par],
                                    gath_out.at[pl.ds(off, GCH)], semw[par])
    for wb in wbs:
        if wb is not None:
            wb.wait()


def _sc_gather(table, gidx):
    nj = gidx.shape[0]
    per_w = nj // NW
    mesh = plsc.VectorSubcoreMesh(core_axis_name="c", subcore_axis_name="s",
                                  num_cores=NC, num_subcores=NS)
    kern = pl.kernel(
        functools.partial(_sc_gather_body, per_w=per_w),
        out_type=jax.ShapeDtypeStruct((nj, D), jnp.float32),
        mesh=mesh,
        compiler_params=pltpu.CompilerParams(use_tc_tiling_on_sc=True,
                                             needs_layout_passes=False),
        scratch_types=[
            pltpu.VMEM((per_w,), jnp.int32),
            pltpu.VMEM((2, GCH, D), jnp.float32),
            pltpu.SemaphoreType.DMA,
            pltpu.SemaphoreType.DMA,
            pltpu.SemaphoreType.DMA,
            pltpu.SemaphoreType.DMA,
        ],
    )
    return kern(table, gidx)


def _sc_body(jc, sidx, acc_out,
             sidxv, rows2, acc, cnt_sm, *,
             per_w, sj, nbatch_per_core, segs_per_tile):
    c = lax.axis_index("c")
    s = lax.axis_index("s")
    i32 = jnp.int32

    tiles_per_batch = NS // nbatch_per_core
    b = nbatch_per_core * c + s // tiles_per_batch
    band = (s % tiles_per_batch) * segs_per_tile
    lo = band
    hi = band + segs_per_tile
    bbase = pl.multiple_of(b * sj, sj)

    zero = jnp.zeros((16,), jnp.float32)

    def _zrow(i, carry):
        for cb in range(ACCW // 16):
            acc[i, pl.ds(cb * 16, 16)] = zero
        return carry

    lax.fori_loop(0, segs_per_tile, _zrow, 0)

    pltpu.sync_copy(sidx.at[pl.ds(bbase, sj)], sidxv.at[pl.ds(0, sj)])

    lane_iota = lax.iota(i32, 16)

    cnt_sm[0] = 0
    cnt_sm[1] = 0

    def _count(i, carry):
        v = sidxv[pl.ds(i * 16, 16)]
        for j, bound in ((0, lo), (1, hi)):
            blt = v < bound
            all_lt = jnp.all(blt)
            any_lt = jnp.any(blt)

            @pl.when(all_lt)
            def _(j=j):
                cnt_sm[j] = cnt_sm[j] + 16

            @pl.when(any_lt & jnp.logical_not(all_lt))
            def _(j=j, blt=blt):
                t = cnt_sm[j]
                for lane in range(16):
                    t = t + jnp.where(
                        jnp.any(blt & (lane_iota == lane)), 1, 0).astype(i32)
                cnt_sm[j] = t
        return carry

    lax.fori_loop(0, sj // 16, _count, 0)
    start = cnt_sm[0]
    end = cnt_sm[1]
    start16 = (start // 16) * 16

    def _lane_splat(v, lane):
        idx = jnp.broadcast_to(lane.astype(i32), (16,))
        dnums = lax.GatherDimensionNumbers(
            offset_dims=(), collapsed_slice_dims=(0,), start_index_map=(0,))
        return lax.gather(v, idx[:, None], dnums, (1,),
                          mode=lax.GatherScatterMode.PROMISE_IN_BOUNDS)

    one0 = jnp.where(lane_iota == 0, 1.0, 0.0)

    def _chunk(k, carry):
        p0 = pl.multiple_of(start16 + k * SCH, 16)
        pltpu.sync_copy(jc.at[pl.ds(bbase + p0, SCH)], rows2)
        for r in range(SCH):
            p = p0 + r

            @pl.when((p >= start) & (p < end))
            def _(p=p, r=r):
                v = sidxv[pl.ds((p // 16) * 16, 16)]
                row = _lane_splat(v, p % 16) - lo
                for cb in range(D // 16):
                    plsc.addupdate_scatter(
                        acc, [row, cb * 16 + lane_iota],
                        rows2[r, pl.ds(cb * 16, 16)])
                plsc.addupdate_scatter(acc, [row, D + lane_iota], one0)
        return carry

    nch = (end - start16 + SCH - 1) // SCH
    lax.fori_loop(0, nch, _chunk, 0)

    out_off = pl.multiple_of(
        b * (segs_per_tile * tiles_per_batch) + band, segs_per_tile)
    pltpu.sync_copy(acc, acc_out.at[pl.ds(out_off, segs_per_tile)])


def _sc_segsum(jc, sidx, nseg, sj):
    nj = sidx.shape[0]
    per_w = nj // NW
    nbatch_per_core = (nj // sj) // NC
    segs_per_tile = nseg // (nj // sj) // (NS // nbatch_per_core)
    mesh = plsc.VectorSubcoreMesh(core_axis_name="c", subcore_axis_name="s",
                                  num_cores=NC, num_subcores=NS)
    kern = pl.kernel(
        functools.partial(_sc_body, per_w=per_w, sj=sj,
                          nbatch_per_core=nbatch_per_core,
                          segs_per_tile=segs_per_tile),
        out_type=jax.ShapeDtypeStruct((nseg, ACCW), jnp.float32),
        mesh=mesh,
        compiler_params=pltpu.CompilerParams(use_tc_tiling_on_sc=True,
                                             needs_layout_passes=False),
        scratch_types=[
            pltpu.VMEM((sj + 32,), jnp.int32),
            pltpu.VMEM((SCH, D), jnp.float32),
            pltpu.VMEM((segs_per_tile, ACCW), jnp.float32),
            pltpu.SMEM((2,), jnp.int32),
        ],
    )
    return kern(jc, sidx)


def kernel(jamo_features, syllable_features, syllable_indices,
           W1a, b1a, W2a, b2a, W1b, b1b, W2b, b2b, g1, beta1, g2, beta2):
    B, SJ, _ = jamo_features.shape
    _, SS, _ = syllable_features.shape
    nj = B * SJ
    nseg = B * SS

    jamo2 = jamo_features.reshape(nj, D)
    syll2 = syllable_features.reshape(nseg, D)

    table = _mlp(syll2, W1a, b1a, W2a, b2a, extra_zero_blocks=1)
    jc = _mlp(jamo2, W1b, b1b, W2b, b2b, extra_zero_blocks=1)

    si = syllable_indices.astype(jnp.int32)
    valid = (si >= 1) & (si <= SS)
    brow = (jnp.arange(B, dtype=jnp.int32) * SS)[:, None]
    gidx = jnp.where(valid, brow + si - 1, nseg).reshape(nj)
    sidx = jnp.where(valid, si - 1, -1).reshape(nj)

    gath = _sc_gather(table, gidx)
    acc = _sc_segsum(jc, sidx, nseg, SJ)

    out1 = _fin_jamo(jamo2, gath, g1, beta1)
    out2 = _fin_syll(acc, syll2, g2, beta2)
    return (out1.reshape(B, SJ, D), out2.reshape(B, SS, D))


# speedup vs baseline: 1.5527x; 1.0010x over previous
import functools

import jax
import jax.numpy as jnp
from jax import lax
from jax.experimental import pallas as pl
from jax.experimental.pallas import tpu as pltpu
from jax.experimental.pallas import tpu_sc as plsc

D = 768
ACCW = 896
NC = 2
NS = 16
NW = NC * NS
GCH = 16
SCH = 32


def _mlp_body(x_ref, w1_ref, b1_ref, w2_ref, b2_ref, o_ref, *, nblk):
    i = pl.program_id(0)

    @pl.when(i < nblk)
    def _():
        bf16 = jnp.bfloat16
        x = x_ref[...].astype(bf16)
        h = lax.dot_general(x, w1_ref[...].astype(bf16),
                            (((1,), (1,)), ((), ())),
                            preferred_element_type=jnp.float32)
        h = h + b1_ref[...]
        h = 0.5 * h * (1.0 + lax.erf(h * 0.7071067811865476))
        o = lax.dot_general(h.astype(bf16), w2_ref[...].astype(bf16),
                            (((1,), (1,)), ((), ())),
                            preferred_element_type=jnp.float32)
        o_ref[...] = o + b2_ref[...]

    @pl.when(i >= nblk)
    def _():
        o_ref[...] = jnp.zeros_like(o_ref)


def _mlp(x, w1, b1, w2, b2, extra_zero_blocks=0, blk=256):
    n = x.shape[0]
    nblk = n // blk
    grid = (nblk + extra_zero_blocks,)
    out = pl.pallas_call(
        functools.partial(_mlp_body, nblk=nblk),
        grid=grid,
        in_specs=[
            pl.BlockSpec((blk, D), lambda i: (jnp.minimum(i, nblk - 1), 0)),
            pl.BlockSpec((D, D), lambda i: (0, 0)),
            pl.BlockSpec((1, D), lambda i: (0, 0)),
            pl.BlockSpec((D, D), lambda i: (0, 0)),
            pl.BlockSpec((1, D), lambda i: (0, 0)),
        ],
        out_specs=pl.BlockSpec((blk, D), lambda i: (i, 0)),
        out_shape=jax.ShapeDtypeStruct(((nblk + extra_zero_blocks) * blk, D),
                                       jnp.float32),
    )(x, w1, b1.reshape(1, D), w2, b2.reshape(1, D))
    return out


def _ln(x, g, b):
    mu = jnp.mean(x, axis=-1, keepdims=True)
    var = jnp.mean((x - mu) ** 2, axis=-1, keepdims=True)
    return (x - mu) * lax.rsqrt(var + 1e-5) * g + b


def _fin_jamo_body(jamo_ref, gath_ref, g_ref, b_ref, o_ref):
    x = jamo_ref[...] + gath_ref[...]
    o_ref[...] = _ln(x, g_ref[...], b_ref[...])


def _fin_jamo(jamo, gath, g, b, blk=256):
    n = jamo.shape[0]
    return pl.pallas_call(
        _fin_jamo_body,
        grid=(n // blk,),
        in_specs=[
            pl.BlockSpec((blk, D), lambda i: (i, 0)),
            pl.BlockSpec((blk, D), lambda i: (i, 0)),
            pl.BlockSpec((1, D), lambda i: (0, 0)),
            pl.BlockSpec((1, D), lambda i: (0, 0)),
        ],
        out_specs=pl.BlockSpec((blk, D), lambda i: (i, 0)),
        out_shape=jax.ShapeDtypeStruct((n, D), jnp.float32),
    )(jamo, gath, g.reshape(1, D), b.reshape(1, D))


def _fin_syll_body(acc_ref, syll_ref, g_ref, b_ref, o_ref):
    a = acc_ref[...]
    sums = a[:, :D]
    cnt = jnp.sum(a[:, D:], axis=1, keepdims=True)
    mean = jnp.where(cnt > 0, sums / jnp.maximum(cnt, 1.0), 0.0)
    x = syll_ref[...] + mean
    o_ref[...] = _ln(x, g_ref[...], b_ref[...])


def _fin_syll(acc, syll, g, b, blk=256):
    n = syll.shape[0]
    return pl.pallas_call(
        _fin_syll_body,
        grid=(n // blk,),
        in_specs=[
            pl.BlockSpec((blk, ACCW), lambda i: (i, 0)),
            pl.BlockSpec((blk, D), lambda i: (i, 0)),
            pl.BlockSpec((1, D), lambda i: (0, 0)),
            pl.BlockSpec((1, D), lambda i: (0, 0)),
        ],
        out_specs=pl.BlockSpec((blk, D), lambda i: (i, 0)),
        out_shape=jax.ShapeDtypeStruct((n, D), jnp.float32),
    )(acc, syll, g.reshape(1, D), b.reshape(1, D))


def _sc_gather_body(table, gidx, gath_out,
                    idx256, rowsg, semg0, semg1, semw0, semw1, *, per_w):
    c = lax.axis_index("c")
    s = lax.axis_index("s")
    wid = c * NS + s
    base = pl.multiple_of(wid * per_w, per_w)
    semg = [semg0, semg1]
    semw = [semw0, semw1]

    ngc = per_w // GCH
    pltpu.sync_copy(gidx.at[pl.ds(base, per_w)], idx256)

    def _gstart(ch, par):
        return pltpu.async_copy(table.at[idx256.at[pl.ds(ch * GCH, GCH)]],
                                rowsg.at[par], semg[par])

    wbs = [None, None]
    cur = _gstart(0, 0)
    for ch in range(ngc):
        par = ch % 2
        cur.wait()
        if ch + 1 < ngc:
            if wbs[1 - par] is not None:
                wbs[1 - par].wait()
            cur = _gstart(ch + 1, 1 - par)
        off = pl.multiple_of(base + ch * GCH, GCH)
        wbs[par] = pltpu.async_copy(rowsg.at[par],
                                    gath_out.at[pl.ds(off, GCH)], semw[par])
    for wb in wbs:
        if wb is not None:
            wb.wait()


def _sc_gather(table, gidx):
    nj = gidx.shape[0]
    per_w = nj // NW
    mesh = plsc.VectorSubcoreMesh(core_axis_name="c", subcore_axis_name="s",
                                  num_cores=NC, num_subcores=NS)
    kern = pl.kernel(
        functools.partial(_sc_gather_body, per_w=per_w),
        out_type=jax.ShapeDtypeStruct((nj, D), jnp.float32),
        mesh=mesh,
        compiler_params=pltpu.CompilerParams(use_tc_tiling_on_sc=True,
                                             needs_layout_passes=False),
        scratch_types=[
            pltpu.VMEM((per_w,), jnp.int32),
            pltpu.VMEM((2, GCH, D), jnp.float32),
            pltpu.SemaphoreType.DMA,
            pltpu.SemaphoreType.DMA,
            pltpu.SemaphoreType.DMA,
            pltpu.SemaphoreType.DMA,
        ],
    )
    return kern(table, gidx)


def _sc_body(jc, sidx, acc_out,
             sidxv, rows2, acc, cnt_sm, sem0, sem1, *,
             per_w, sj, nbatch_per_core, segs_per_tile):
    c = lax.axis_index("c")
    s = lax.axis_index("s")
    i32 = jnp.int32

    tiles_per_batch = NS // nbatch_per_core
    b = nbatch_per_core * c + s // tiles_per_batch
    band = (s % tiles_per_batch) * segs_per_tile
    lo = band
    hi = band + segs_per_tile
    bbase = pl.multiple_of(b * sj, sj)

    zero = jnp.zeros((16,), jnp.float32)

    def _zrow(i, carry):
        for cb in range(ACCW // 16):
            acc[i, pl.ds(cb * 16, 16)] = zero
        return carry

    lax.fori_loop(0, segs_per_tile, _zrow, 0)

    pltpu.sync_copy(sidx.at[pl.ds(bbase, sj)], sidxv.at[pl.ds(0, sj)])

    lane_iota = lax.iota(i32, 16)

    cnt_sm[0] = 0
    cnt_sm[1] = 0

    def _count(i, carry):
        v = sidxv[pl.ds(i * 16, 16)]
        for j, bound in ((0, lo), (1, hi)):
            blt = v < bound
            all_lt = jnp.all(blt)
            any_lt = jnp.any(blt)

            @pl.when(all_lt)
            def _(j=j):
                cnt_sm[j] = cnt_sm[j] + 16

            @pl.when(any_lt & jnp.logical_not(all_lt))
            def _(j=j, blt=blt):
                t = cnt_sm[j]
                for lane in range(16):
                    t = t + jnp.where(
                        jnp.any(blt & (lane_iota == lane)), 1, 0).astype(i32)
                cnt_sm[j] = t
        return carry

    lax.fori_loop(0, sj // 16, _count, 0)
    start = cnt_sm[0]
    end = cnt_sm[1]
    start16 = (start // 16) * 16

    def _lane_splat(v, lane):
        idx = jnp.broadcast_to(lane.astype(i32), (16,))
        dnums = lax.GatherDimensionNumbers(
            offset_dims=(), collapsed_slice_dims=(0,), start_index_map=(0,))
        return lax.gather(v, idx[:, None], dnums, (1,),
                          mode=lax.GatherScatterMode.PROMISE_IN_BOUNDS)

    one0 = jnp.where(lane_iota == 0, 1.0, 0.0)
    sems = [sem0, sem1]

    def _load(k, par):
        p0 = pl.multiple_of(start16 + k * SCH, 16)
        pltpu.async_copy(jc.at[pl.ds(bbase + p0, SCH)], rows2.at[par],
                         sems[par])

    def _drain(par):
        pltpu.make_async_copy(jc.at[pl.ds(0, SCH)], rows2.at[par],
                              sems[par]).wait()

    def _proc(k, par):
        p0 = pl.multiple_of(start16 + k * SCH, 16)
        for r in range(SCH):
            p = p0 + r

            @pl.when((p >= start) & (p < end))
            def _(p=p, r=r):
                v = sidxv[pl.ds((p // 16) * 16, 16)]
                row = _lane_splat(v, p % 16) - lo
                for cb in range(D // 16):
                    plsc.addupdate_scatter(
                        acc, [row, cb * 16 + lane_iota],
                        rows2[par, r, pl.ds(cb * 16, 16)])
                plsc.addupdate_scatter(acc, [row, D + lane_iota], one0)

    nch = (end - start16 + SCH - 1) // SCH
    nch2 = (nch + 1) // 2
    _load(0, 0)

    def _pair(k2, carry):
        ka = 2 * k2
        _load(ka + 1, 1)
        _drain(0)
        _proc(ka, 0)
        _load(ka + 2, 0)
        _drain(1)
        _proc(ka + 1, 1)
        return carry

    lax.fori_loop(0, nch2, _pair, 0)
    _drain(0)

    out_off = pl.multiple_of(
        b * (segs_per_tile * tiles_per_batch) + band, segs_per_tile)
    pltpu.sync_copy(acc, acc_out.at[pl.ds(out_off, segs_per_tile)])


def _sc_segsum(jc, sidx, nseg, sj):
    nj = sidx.shape[0]
    per_w = nj // NW
    nbatch_per_core = (nj // sj) // NC
    segs_per_tile = nseg // (nj // sj) // (NS // nbatch_per_core)
    mesh = plsc.VectorSubcoreMesh(core_axis_name="c", subcore_axis_name="s",
                                  num_cores=NC, num_subcores=NS)
    kern = pl.kernel(
        functools.partial(_sc_body, per_w=per_w, sj=sj,
                          nbatch_per_core=nbatch_per_core,
                          segs_per_tile=segs_per_tile),
        out_type=jax.ShapeDtypeStruct((nseg, ACCW), jnp.float32),
        mesh=mesh,
        compiler_params=pltpu.CompilerParams(use_tc_tiling_on_sc=True,
                                             needs_layout_passes=False),
        scratch_types=[
            pltpu.VMEM((sj + 32,), jnp.int32),
            pltpu.VMEM((2, SCH, D), jnp.float32),
            pltpu.VMEM((segs_per_tile, ACCW), jnp.float32),
            pltpu.SMEM((2,), jnp.int32),
            pltpu.SemaphoreType.DMA,
            pltpu.SemaphoreType.DMA,
        ],
    )
    return kern(jc, sidx)


def kernel(jamo_features, syllable_features, syllable_indices,
           W1a, b1a, W2a, b2a, W1b, b1b, W2b, b2b, g1, beta1, g2, beta2):
    B, SJ, _ = jamo_features.shape
    _, SS, _ = syllable_features.shape
    nj = B * SJ
    nseg = B * SS

    jamo2 = jamo_features.reshape(nj, D)
    syll2 = syllable_features.reshape(nseg, D)

    jc = _mlp(jamo2, W1b, b1b, W2b, b2b, extra_zero_blocks=1)
    table = _mlp(syll2, W1a, b1a, W2a, b2a, extra_zero_blocks=1)

    si = syllable_indices.astype(jnp.int32)
    valid = (si >= 1) & (si <= SS)
    brow = (jnp.arange(B, dtype=jnp.int32) * SS)[:, None]
    gidx = jnp.where(valid, brow + si - 1, nseg).reshape(nj)
    sidx = jnp.where(valid, si - 1, -1).reshape(nj)

    gath = _sc_gather(table, gidx)
    acc = _sc_segsum(jc, sidx, nseg, SJ)

    out1 = _fin_jamo(jamo2, gath, g1, beta1)
    out2 = _fin_syll(acc, syll2, g2, beta2)
    return (out1.reshape(B, SJ, D), out2.reshape(B, SS, D))
